# initial kernel scaffold (unmeasured)
import jax
import jax.numpy as jnp
from jax import lax
from jax.experimental import pallas as pl
from jax.experimental.pallas import tpu as pltpu


def kernel(
    x,
):
    def body(*refs):
        pass

    out_shape = jax.ShapeDtypeStruct(..., jnp.float32)
    return pl.pallas_call(body, out_shape=out_shape)(...)


# baseline (device time: 572134 ns/iter reference)
import jax
import jax.numpy as jnp
from jax import lax
from jax.experimental import pallas as pl
from jax.experimental.pallas import tpu as pltpu


def kernel(x):
    x16 = x.astype(jnp.bfloat16)
    m, n = x16.shape
    half = n // 2

    def body(x_ref, out_ref, local_sem, send_sem, recv_sem):
        my_p = lax.axis_index("x")
        my_y = lax.axis_index("y")
        my_z = lax.axis_index("z")
        peer = 1 - my_p

        local = pltpu.make_async_copy(
            x_ref.at[:, pl.ds(my_p * half, half)],
            out_ref.at[pl.ds(my_p * m, m), :],
            local_sem,
        )
        local.start()

        rdma = pltpu.make_async_remote_copy(
            src_ref=x_ref.at[:, pl.ds(peer * half, half)],
            dst_ref=out_ref.at[pl.ds(my_p * m, m), :],
            send_sem=send_sem,
            recv_sem=recv_sem,
            device_id=(peer, my_y, my_z),
            device_id_type=pl.DeviceIdType.MESH,
        )
        rdma.start()

        local.wait()
        rdma.wait()

    return pl.pallas_call(
        body,
        out_shape=jax.ShapeDtypeStruct((2 * m, half), jnp.bfloat16),
        in_specs=[pl.BlockSpec(memory_space=pltpu.MemorySpace.HBM)],
        out_specs=pl.BlockSpec(memory_space=pltpu.MemorySpace.HBM),
        scratch_shapes=[
            pltpu.SemaphoreType.DMA,
            pltpu.SemaphoreType.DMA,
            pltpu.SemaphoreType.DMA,
        ],
    )(x16)
